# Initial kernel scaffold; baseline (speedup 1.0000x reference)
import jax
import jax.numpy as jnp
from jax.experimental import pallas as pl


def kernel(x, Wr, W1, W2):
    raise NotImplementedError("write your pallas kernel here")


# speedup vs baseline: 8.8769x; 8.8769x over previous
import jax
import jax.numpy as jnp
from jax.experimental import pallas as pl
from jax.experimental.pallas import tpu as pltpu

_E = 64
_D = 48
_TM = 256


def _moe_dense_kernel(x_ref, wr_ref, w1_ref, w2_ref, o_ref):
    x = x_ref[...]
    logits = jax.lax.dot_general(
        x, wr_ref[...], (((1,), (1,)), ((), ())),
        preferred_element_type=jnp.float32,
        precision=jax.lax.Precision.DEFAULT)
    m = jnp.max(logits, axis=-1, keepdims=True)
    lane = jax.lax.broadcasted_iota(jnp.int32, logits.shape, 1)
    eid = jnp.min(jnp.where(logits == m, lane, _E), axis=-1)

    h = jax.lax.dot_general(
        x, w1_ref[...], (((1,), (0,)), ((), ())),
        preferred_element_type=jnp.float32,
        precision=jax.lax.Precision.DEFAULT)
    h = jnp.maximum(h, 0.0)
    h = h * h
    col = jax.lax.broadcasted_iota(jnp.int32, h.shape, 1)
    lo = (eid * _D)[:, None]
    g = jnp.where((col >= lo) & (col < lo + _D), h, 0.0)

    o_ref[...] = jax.lax.dot_general(
        g, w2_ref[...], (((1,), (0,)), ((), ())),
        preferred_element_type=jnp.float32,
        precision=jax.lax.Precision.DEFAULT)


def kernel(x, Wr, W1, W2):
    B, T, C = x.shape
    N = B * T
    x_flat = x.reshape(N, C)
    E, _, D = W1.shape
    w1cat = W1.transpose(1, 0, 2).reshape(C, E * D)
    w2cat = W2.reshape(E * D, C)

    out = pl.pallas_call(
        _moe_dense_kernel,
        grid=(N // _TM,),
        in_specs=[
            pl.BlockSpec((_TM, C), lambda i: (i, 0)),
            pl.BlockSpec((E, C), lambda i: (0, 0)),
            pl.BlockSpec((C, E * D), lambda i: (0, 0)),
            pl.BlockSpec((E * D, C), lambda i: (0, 0)),
        ],
        out_specs=pl.BlockSpec((_TM, C), lambda i: (i, 0)),
        out_shape=jax.ShapeDtypeStruct((N, C), jnp.float32),
    )(x_flat, Wr, w1cat, w2cat)
    return out.reshape(B, T, C)
